# baseline (device time: 12665 ns/iter reference)
import jax
import jax.numpy as jnp
from jax import lax
from jax.experimental import pallas as pl
from jax.experimental.pallas import tpu as pltpu

C = 2
R = 4


def kernel(x):
    m, n = x.shape
    bm = m // R
    nc = n // C

    def body(x_ref, out_ref, comm_ref, send_sems, recv_sems):
        c = pl.program_id(0)
        r = pl.program_id(1)
        my_x = lax.axis_index("x")
        my_y = lax.axis_index("y")
        peer = (1 - my_x, my_y)
        barrier_sem = pltpu.get_barrier_semaphore()

        @pl.when((c == 0) & (r == 0))
        def _():
            pl.semaphore_signal(
                barrier_sem, inc=1, device_id=peer,
                device_id_type=pl.DeviceIdType.MESH,
            )

        blk = jnp.max(x_ref[:, :], axis=0, keepdims=True)

        for cc in range(C):
            csl = pl.ds(cc * nc, nc)

            @pl.when((c == cc) & (r == 0))
            def _():
                out_ref[:, csl] = blk

            @pl.when((c == cc) & (r > 0))
            def _():
                out_ref[:, csl] = jnp.maximum(out_ref[:, csl], blk)

            @pl.when((c == cc) & (r == R - 1))
            def _():
                if cc == 0:
                    pl.semaphore_wait(barrier_sem, 1)
                rdma = pltpu.make_async_remote_copy(
                    src_ref=out_ref.at[:, csl],
                    dst_ref=comm_ref.at[cc],
                    send_sem=send_sems.at[cc],
                    recv_sem=recv_sems.at[cc],
                    device_id=peer,
                    device_id_type=pl.DeviceIdType.MESH,
                )
                rdma.start()

        @pl.when((c == C - 1) & (r == R - 1))
        def _():
            for cc in range(C):
                csl = pl.ds(cc * nc, nc)
                rdma = pltpu.make_async_remote_copy(
                    src_ref=out_ref.at[:, csl],
                    dst_ref=comm_ref.at[cc],
                    send_sem=send_sems.at[cc],
                    recv_sem=recv_sems.at[cc],
                    device_id=peer,
                    device_id_type=pl.DeviceIdType.MESH,
                )
                rdma.wait()
                out_ref[:, csl] = jnp.maximum(out_ref[:, csl], comm_ref[cc])

    return pl.pallas_call(
        body,
        grid=(C, R),
        out_shape=jax.ShapeDtypeStruct((1, n), x.dtype),
        in_specs=[pl.BlockSpec((bm, nc), lambda c, r: (r, c))],
        out_specs=pl.BlockSpec((1, n), lambda c, r: (0, 0)),
        scratch_shapes=[
            pltpu.VMEM((C, 1, nc), x.dtype),
            pltpu.SemaphoreType.DMA((C,)),
            pltpu.SemaphoreType.DMA((C,)),
        ],
        compiler_params=pltpu.CompilerParams(collective_id=0),
    )(x)


# device time: 10745 ns/iter; 1.1787x vs baseline; 1.1787x over previous
import jax
import jax.numpy as jnp
from jax import lax
from jax.experimental import pallas as pl
from jax.experimental.pallas import tpu as pltpu


def kernel(x):
    m, n = x.shape

    def body(x_ref, out_ref, comm_ref, send_sem, recv_sem):
        my_x = lax.axis_index("x")
        my_y = lax.axis_index("y")
        peer = (1 - my_x, my_y)

        barrier_sem = pltpu.get_barrier_semaphore()
        pl.semaphore_signal(
            barrier_sem, inc=1, device_id=peer,
            device_id_type=pl.DeviceIdType.MESH,
        )

        out_ref[:, :] = jnp.max(x_ref[:, :], axis=0, keepdims=True)

        pl.semaphore_wait(barrier_sem, 1)

        rdma = pltpu.make_async_remote_copy(
            src_ref=out_ref,
            dst_ref=comm_ref,
            send_sem=send_sem,
            recv_sem=recv_sem,
            device_id=peer,
            device_id_type=pl.DeviceIdType.MESH,
        )
        rdma.start()
        rdma.wait_recv()
        out_ref[:, :] = jnp.maximum(out_ref[:, :], comm_ref[:, :])
        rdma.wait_send()

    return pl.pallas_call(
        body,
        out_shape=jax.ShapeDtypeStruct((1, n), x.dtype),
        in_specs=[pl.BlockSpec(memory_space=pltpu.VMEM)],
        out_specs=pl.BlockSpec(memory_space=pltpu.VMEM),
        scratch_shapes=[
            pltpu.VMEM((1, n), x.dtype),
            pltpu.SemaphoreType.DMA,
            pltpu.SemaphoreType.DMA,
        ],
        compiler_params=pltpu.CompilerParams(collective_id=0),
    )(x)
